# baseline (speedup 1.0000x reference)
import jax
import jax.numpy as jnp
from jax.experimental import pallas as pl

_BM = 400
_BH = _BM // 2


def _fused_body(xblk_ref, adja_ref, adjb_ref, x_ref, w_ref, o_ref):
    x = x_ref[...]
    f = x.shape[1]
    w_top = w_ref[:f, :]
    w_bot = w_ref[f:, :]
    self_term = jnp.dot(xblk_ref[...], w_top,
                        preferred_element_type=jnp.float32)
    adja = adja_ref[...]
    adjb = adjb_ref[...]
    dega = jnp.sum(adja, axis=1, keepdims=True) + 1e-6
    degb = jnp.sum(adjb, axis=1, keepdims=True) + 1e-6
    ha = jnp.dot(adja, x, preferred_element_type=jnp.float32)
    hb = jnp.dot(adjb, x, preferred_element_type=jnp.float32)
    o_ref[:_BH, :] = self_term[:_BH, :] + jnp.dot(
        ha / dega, w_bot, preferred_element_type=jnp.float32)
    o_ref[_BH:, :] = self_term[_BH:, :] + jnp.dot(
        hb / degb, w_bot, preferred_element_type=jnp.float32)


def kernel(input, adj, W):
    n, f = input.shape
    out_f = W.shape[1]
    grid = (pl.cdiv(n, _BM),)
    return pl.pallas_call(
        _fused_body,
        grid=grid,
        in_specs=[
            pl.BlockSpec((_BM, f), lambda i: (i, 0)),
            pl.BlockSpec((_BH, n), lambda i: (2 * i, 0)),
            pl.BlockSpec((_BH, n), lambda i: (2 * i + 1, 0)),
            pl.BlockSpec((n, f), lambda i: (0, 0)),
            pl.BlockSpec(W.shape, lambda i: (0, 0)),
        ],
        out_specs=pl.BlockSpec((_BM, out_f), lambda i: (i, 0)),
        out_shape=jax.ShapeDtypeStruct((n, out_f), jnp.float32),
    )(input, adj, adj, input, W)


# speedup vs baseline: 1.0172x; 1.0172x over previous
import jax
import jax.numpy as jnp
from jax.experimental import pallas as pl
from jax.experimental.pallas import tpu as pltpu

_BM = 400


def _fused_body(xblk_ref, adj_ref, x_ref, w_ref, o_ref):
    adj = adj_ref[...]
    deg = jnp.sum(adj, axis=1, keepdims=True) + 1e-6
    h = jnp.dot(adj, x_ref[...], preferred_element_type=jnp.float32)
    f = x_ref.shape[1]
    w_top = w_ref[:f, :]
    w_bot = w_ref[f:, :]
    self_term = jnp.dot(xblk_ref[...], w_top,
                        preferred_element_type=jnp.float32)
    agg_term = jnp.dot(h / deg, w_bot, preferred_element_type=jnp.float32)
    o_ref[...] = self_term + agg_term


def kernel(input, adj, W):
    n, f = input.shape
    out_f = W.shape[1]
    grid = (pl.cdiv(n, _BM),)
    return pl.pallas_call(
        _fused_body,
        grid=grid,
        in_specs=[
            pl.BlockSpec((_BM, f), lambda i: (i, 0)),
            pl.BlockSpec((_BM, n), lambda i: (i, 0)),
            pl.BlockSpec((n, f), lambda i: (0, 0)),
            pl.BlockSpec(W.shape, lambda i: (0, 0)),
        ],
        out_specs=pl.BlockSpec((_BM, out_f), lambda i: (i, 0)),
        out_shape=jax.ShapeDtypeStruct((n, out_f), jnp.float32),
        compiler_params=pltpu.CompilerParams(
            dimension_semantics=("parallel",)),
    )(input, adj, input, W)


# speedup vs baseline: 1.0296x; 1.0123x over previous
import jax
import jax.numpy as jnp
from jax.experimental import pallas as pl

_BM = 400


def _fused_body(adj_ref, x_ref, w_ref, o_ref):
    adj = adj_ref[...]
    deg = jnp.sum(adj, axis=1, keepdims=True) + 1e-6
    h = jnp.dot(adj, x_ref[...], preferred_element_type=jnp.float32)
    f = x_ref.shape[1]
    w_top = w_ref[:f, :]
    w_bot = w_ref[f:, :]
    xblk = x_ref[pl.ds(pl.program_id(0) * _BM, _BM), :]
    self_term = jnp.dot(xblk, w_top, preferred_element_type=jnp.float32)
    agg_term = jnp.dot(h / deg, w_bot, preferred_element_type=jnp.float32)
    o_ref[...] = self_term + agg_term


def kernel(input, adj, W):
    n, f = input.shape
    out_f = W.shape[1]
    grid = (pl.cdiv(n, _BM),)
    return pl.pallas_call(
        _fused_body,
        grid=grid,
        in_specs=[
            pl.BlockSpec((_BM, n), lambda i: (i, 0)),
            pl.BlockSpec((n, f), lambda i: (0, 0)),
            pl.BlockSpec(W.shape, lambda i: (0, 0)),
        ],
        out_specs=pl.BlockSpec((_BM, out_f), lambda i: (i, 0)),
        out_shape=jax.ShapeDtypeStruct((n, out_f), jnp.float32),
    )(adj, input, W)
